# Initial kernel scaffold; baseline (speedup 1.0000x reference)
import jax
import jax.numpy as jnp
from jax.experimental import pallas as pl


def kernel(x, edge_index, W1, b1, W2, b2):
    raise NotImplementedError("write your pallas kernel here")


# speedup vs baseline: 3.5144x; 3.5144x over previous
import functools

import jax
import jax.numpy as jnp
from jax import lax
from jax.experimental import pallas as pl
from jax.experimental.pallas import tpu as pltpu
from jax.experimental.pallas import tpu_sc as plsc

N = 10000
E = 160000
D_IN = 2048
D_HID = 1024
D_OUT = 512

NP = 10240
BIN = N
NSUB = 16
NCORE = 2
EPT = E // NSUB
EB = 128
NB = NP // EB
STRIPE = NP // NSUB
MT = 512

def _sc_mesh():
    return plsc.VectorSubcoreMesh(core_axis_name="c", subcore_axis_name="s")


def _deg_body(dsti, ones_h, zeros_h, deg_out, dst_v, ones_v, dacc):
    c = lax.axis_index("c")
    s = lax.axis_index("s")
    nb_half = NB // 2
    pltpu.sync_copy(dsti.at[s], dst_v)
    pltpu.sync_copy(ones_h, ones_v)
    pltpu.sync_copy(zeros_h, dacc.at[pl.ds(s * STRIPE, STRIPE)])
    plsc.subcore_barrier()

    def body(b, carry):
        pltpu.sync_copy(ones_v, dacc.at[dst_v.at[b]], add=True)
        return carry

    lax.fori_loop(c * nb_half, (c + 1) * nb_half, body, 0)
    plsc.subcore_barrier()
    pltpu.sync_copy(dacc.at[pl.ds(s * STRIPE, STRIPE)],
                    deg_out.at[pl.ds(c * NP + s * STRIPE, STRIPE)])


def _sc_degree(dst_idx, ones128, zeros128):
    f = pl.kernel(
        _deg_body,
        out_type=jax.ShapeDtypeStruct((NCORE * NP, EB), jnp.float32),
        mesh=_sc_mesh(),
        scratch_types=[
            pltpu.VMEM((NB, EB), jnp.int32),
            pltpu.VMEM((EB, EB), jnp.float32),
            pltpu.VMEM_SHARED((NP, EB), jnp.float32),
        ],
    )
    return f(dst_idx, ones128, zeros128).reshape(NCORE, NP, EB)


def _agg_body(nchunk, g_flat, srci, dsti, zeros_h, out_flat,
              src_v, dst_v, rows_v, accum, sem):
    c = lax.axis_index("c")
    s = lax.axis_index("s")
    cpc = nchunk // NCORE
    pltpu.sync_copy(dsti.at[s], dst_v)
    for cc in range(cpc):
        chunk = c * cpc + cc
        pltpu.sync_copy(srci.at[chunk, s], src_v)
        pltpu.sync_copy(zeros_h, accum.at[pl.ds(s * STRIPE, STRIPE)])
        plsc.subcore_barrier()

        def body(b, carry):
            pltpu.async_copy(g_flat.at[src_v.at[b]], rows_v, sem).wait()
            pltpu.sync_copy(rows_v, accum.at[dst_v.at[b]], add=True)
            return carry

        lax.fori_loop(0, NB, body, 0)
        plsc.subcore_barrier()
        pltpu.sync_copy(accum.at[pl.ds(s * STRIPE, STRIPE)],
                        out_flat.at[pl.ds(chunk * NP + s * STRIPE, STRIPE)])
        plsc.subcore_barrier()


def _sc_aggregate(nchunk, g_flat, src_idx_shifted, dst_idx, zeros128):
    f = pl.kernel(
        functools.partial(_agg_body, nchunk),
        out_type=jax.ShapeDtypeStruct((nchunk * NP, EB), jnp.float32),
        mesh=_sc_mesh(),
        scratch_types=[
            pltpu.VMEM((NB, EB), jnp.int32),
            pltpu.VMEM((NB, EB), jnp.int32),
            pltpu.VMEM((EB, EB), jnp.float32),
            pltpu.VMEM_SHARED((NP, EB), jnp.float32),
            pltpu.SemaphoreType.DMA,
        ],
    )
    return f(g_flat, src_idx_shifted, dst_idx, zeros128)


def _dinv_of(deg_ref):
    return lax.rsqrt(deg_ref[0, :, 0:1] + deg_ref[1, :, 0:1] + 1.0)


def _mm_body(x_ref, w_ref, deg_ref, o_ref):
    dinv = _dinv_of(deg_ref)
    o_ref[0] = dinv * jnp.dot(x_ref[...], w_ref[...],
                              preferred_element_type=jnp.float32)


def _mm_scaled(x, w, deg, nchunk):
    k = x.shape[1]
    return pl.pallas_call(
        _mm_body,
        grid=(NP // MT, nchunk),
        in_specs=[
            pl.BlockSpec((MT, k), lambda m, c: (m, 0)),
            pl.BlockSpec((k, EB), lambda m, c: (0, c)),
            pl.BlockSpec((NCORE, MT, EB), lambda m, c: (0, m, 0)),
        ],
        out_specs=pl.BlockSpec((1, MT, EB), lambda m, c: (c, m, 0)),
        out_shape=jax.ShapeDtypeStruct((nchunk, NP, EB), jnp.float32),
    )(x, w, deg)


def _ew_body(relu, agg_ref, g_ref, deg_ref, b_ref, o_ref):
    dinv = _dinv_of(deg_ref)
    v = dinv * (agg_ref[0] + g_ref[0]) + b_ref[0]
    o_ref[...] = jnp.maximum(v, 0.0) if relu else v


def _ew_combine(agg_cm, g_cm, deg, b, relu):
    nchunk = agg_cm.shape[0]
    return pl.pallas_call(
        functools.partial(_ew_body, relu),
        grid=(NP // MT, nchunk),
        in_specs=[
            pl.BlockSpec((1, MT, EB), lambda m, c: (c, m, 0)),
            pl.BlockSpec((1, MT, EB), lambda m, c: (c, m, 0)),
            pl.BlockSpec((NCORE, MT, EB), lambda m, c: (0, m, 0)),
            pl.BlockSpec((1, EB), lambda m, c: (0, c)),
        ],
        out_specs=pl.BlockSpec((MT, EB), lambda m, c: (m, c)),
        out_shape=jax.ShapeDtypeStruct((NP, nchunk * EB), jnp.float32),
    )(agg_cm, g_cm, deg, b.reshape(1, -1))


def kernel(x, edge_index, W1, b1, W2, b2):
    src = edge_index[0].astype(jnp.int32)
    dst = edge_index[1].astype(jnp.int32)

    srcr = jnp.pad(src.reshape(NSUB, EPT), ((0, 0), (0, NP - EPT)))
    dstr = jnp.pad(dst.reshape(NSUB, EPT), ((0, 0), (0, NP - EPT)),
                   constant_values=BIN)
    dst_idx = dstr.reshape(NSUB, NB, EB)
    shifts8 = (jnp.arange(8, dtype=jnp.int32) * NP)[:, None, None, None]
    src_idx8 = (srcr.reshape(1, NSUB, NB, EB) + shifts8)
    src_idx4 = src_idx8[:4]

    ones128 = jnp.ones((EB, EB), jnp.float32)
    zeros128 = jnp.zeros((STRIPE, EB), jnp.float32)

    x_pad = jnp.pad(x, ((0, NP - N), (0, 0)))

    deg = _sc_degree(dst_idx, ones128, zeros128)

    g1_cm = _mm_scaled(x_pad, W1, deg, D_HID // EB)
    agg1 = _sc_aggregate(8, g1_cm.reshape(8 * NP, EB), src_idx8, dst_idx,
                         zeros128).reshape(8, NP, EB)
    h1 = _ew_combine(agg1, g1_cm, deg, b1, relu=True)

    g2_cm = _mm_scaled(h1, W2, deg, D_OUT // EB)
    agg2 = _sc_aggregate(4, g2_cm.reshape(4 * NP, EB), src_idx4, dst_idx,
                         zeros128).reshape(4, NP, EB)
    out = _ew_combine(agg2, g2_cm, deg, b2, relu=False)
    return out[:N]


# speedup vs baseline: 3.7207x; 1.0587x over previous
import functools

import jax
import jax.numpy as jnp
from jax import lax
from jax.experimental import pallas as pl
from jax.experimental.pallas import tpu as pltpu
from jax.experimental.pallas import tpu_sc as plsc

N = 10000
E = 160000
D_IN = 2048
D_HID = 1024
D_OUT = 512

NP = 10240
BIN = N
NSUB = 16
NCORE = 2
EPT = E // NSUB
EB = 128
NB = NP // EB
STRIPE = NP // NSUB
MT = 512

def _sc_mesh():
    return plsc.VectorSubcoreMesh(core_axis_name="c", subcore_axis_name="s")


def _deg_body(dsti, ones_h, zeros_h, deg_out, dst_v, ones_v, dacc):
    c = lax.axis_index("c")
    s = lax.axis_index("s")
    nb_half = NB // 2
    pltpu.sync_copy(dsti.at[s], dst_v)
    pltpu.sync_copy(ones_h, ones_v)
    pltpu.sync_copy(zeros_h, dacc.at[pl.ds(s * STRIPE, STRIPE)])
    plsc.subcore_barrier()

    def body(b, carry):
        pltpu.sync_copy(ones_v, dacc.at[dst_v.at[b]], add=True)
        return carry

    lax.fori_loop(c * nb_half, (c + 1) * nb_half, body, 0)
    plsc.subcore_barrier()
    pltpu.sync_copy(dacc.at[pl.ds(s * STRIPE, STRIPE)],
                    deg_out.at[pl.ds(c * NP + s * STRIPE, STRIPE)])


def _sc_degree(dst_idx, ones128, zeros128):
    f = pl.kernel(
        _deg_body,
        out_type=jax.ShapeDtypeStruct((NCORE * NP, EB), jnp.float32),
        mesh=_sc_mesh(),
        scratch_types=[
            pltpu.VMEM((NB, EB), jnp.int32),
            pltpu.VMEM((EB, EB), jnp.float32),
            pltpu.VMEM_SHARED((NP, EB), jnp.float32),
        ],
    )
    return f(dst_idx, ones128, zeros128).reshape(NCORE, NP, EB)


def _agg_body(nchunk, g_flat, combi, zeros_h, out_flat,
              idx_v, rows_a, rows_b, gsa, gsb, accum):
    c = lax.axis_index("c")
    s = lax.axis_index("s")
    cpc = nchunk // NCORE

    nbh = NB // 2

    def chunk_body(cc, carry2):
        chunk = c * cpc + cc
        pltpu.sync_copy(zeros_h, accum.at[pl.ds(s * STRIPE, STRIPE)])
        plsc.subcore_barrier()

        def half_body(h, carryh):
            pltpu.sync_copy(combi.at[chunk, s].at[:, pl.ds(h * nbh, nbh)],
                            idx_v)
            src_v = idx_v.at[0]
            dst_v = idx_v.at[1]

            def body(g, carry):
                b0 = 2 * g
                ga = pltpu.async_copy(g_flat.at[src_v.at[b0]], rows_a, gsa)
                gb = pltpu.async_copy(g_flat.at[src_v.at[b0 + 1]], rows_b,
                                      gsb)
                ga.wait()
                pltpu.sync_copy(rows_a, accum.at[dst_v.at[b0]], add=True)
                gb.wait()
                pltpu.sync_copy(rows_b, accum.at[dst_v.at[b0 + 1]], add=True)
                return carry

            lax.fori_loop(0, nbh // 2, body, 0)
            return carryh

        lax.fori_loop(0, 2, half_body, 0)
        plsc.subcore_barrier()
        pltpu.sync_copy(accum.at[pl.ds(s * STRIPE, STRIPE)],
                        out_flat.at[pl.ds(chunk * NP + s * STRIPE, STRIPE)])
        plsc.subcore_barrier()
        return carry2

    lax.fori_loop(0, cpc, chunk_body, 0)


def _sc_aggregate(nchunk, g_flat, comb_idx, zeros128):
    f = pl.kernel(
        functools.partial(_agg_body, nchunk),
        out_type=jax.ShapeDtypeStruct((nchunk * NP, EB), jnp.float32),
        mesh=_sc_mesh(),
        scratch_types=[
            pltpu.VMEM((2, NB // 2, EB), jnp.int32),
            pltpu.VMEM((EB, EB), jnp.float32),
            pltpu.VMEM((EB, EB), jnp.float32),
            pltpu.SemaphoreType.DMA,
            pltpu.SemaphoreType.DMA,
            pltpu.VMEM_SHARED((NP, EB), jnp.float32),
        ],
    )
    return f(g_flat, comb_idx, zeros128)


def _dinv_of(deg_ref):
    return lax.rsqrt(deg_ref[0, :, 0:1] + deg_ref[1, :, 0:1] + 1.0)


def _mm_body(x_ref, w_ref, deg_ref, o_ref):
    dinv = _dinv_of(deg_ref)
    o_ref[0] = dinv * jnp.dot(x_ref[...], w_ref[...],
                              preferred_element_type=jnp.float32)


def _mm_scaled(x, w, deg, nchunk):
    k = x.shape[1]
    return pl.pallas_call(
        _mm_body,
        grid=(NP // MT, nchunk),
        in_specs=[
            pl.BlockSpec((MT, k), lambda m, c: (m, 0)),
            pl.BlockSpec((k, EB), lambda m, c: (0, c)),
            pl.BlockSpec((NCORE, MT, EB), lambda m, c: (0, m, 0)),
        ],
        out_specs=pl.BlockSpec((1, MT, EB), lambda m, c: (c, m, 0)),
        out_shape=jax.ShapeDtypeStruct((nchunk, NP, EB), jnp.float32),
    )(x, w, deg)


def _ew_body(relu, agg_ref, g_ref, deg_ref, b_ref, o_ref):
    dinv = _dinv_of(deg_ref)
    v = dinv * (agg_ref[0] + g_ref[0]) + b_ref[0]
    o_ref[...] = jnp.maximum(v, 0.0) if relu else v


def _ew_combine(agg_cm, g_cm, deg, b, relu):
    nchunk = agg_cm.shape[0]
    return pl.pallas_call(
        functools.partial(_ew_body, relu),
        grid=(NP // MT, nchunk),
        in_specs=[
            pl.BlockSpec((1, MT, EB), lambda m, c: (c, m, 0)),
            pl.BlockSpec((1, MT, EB), lambda m, c: (c, m, 0)),
            pl.BlockSpec((NCORE, MT, EB), lambda m, c: (0, m, 0)),
            pl.BlockSpec((1, EB), lambda m, c: (0, c)),
        ],
        out_specs=pl.BlockSpec((MT, EB), lambda m, c: (m, c)),
        out_shape=jax.ShapeDtypeStruct((NP, nchunk * EB), jnp.float32),
    )(agg_cm, g_cm, deg, b.reshape(1, -1))


def kernel(x, edge_index, W1, b1, W2, b2):
    src = edge_index[0].astype(jnp.int32)
    dst = edge_index[1].astype(jnp.int32)

    srcr = jnp.pad(src.reshape(NSUB, EPT), ((0, 0), (0, NP - EPT)))
    dstr = jnp.pad(dst.reshape(NSUB, EPT), ((0, 0), (0, NP - EPT)),
                   constant_values=BIN)
    dst_idx = dstr.reshape(NSUB, NB, EB)
    shifts8 = (jnp.arange(8, dtype=jnp.int32) * NP)[:, None, None, None]
    src_idx8 = (srcr.reshape(1, NSUB, NB, EB) + shifts8)
    dst_b = jnp.broadcast_to(dst_idx[None], (8, NSUB, NB, EB))
    comb8 = jnp.stack([src_idx8, dst_b], axis=2)
    comb4 = comb8[:4]

    ones128 = jnp.ones((EB, EB), jnp.float32)
    zeros128 = jnp.zeros((STRIPE, EB), jnp.float32)

    x_pad = jnp.pad(x, ((0, NP - N), (0, 0)))

    deg = _sc_degree(dst_idx, ones128, zeros128)

    g1_cm = _mm_scaled(x_pad, W1, deg, D_HID // EB)
    agg1 = _sc_aggregate(8, g1_cm.reshape(8 * NP, EB), comb8,
                         zeros128).reshape(8, NP, EB)
    h1 = _ew_combine(agg1, g1_cm, deg, b1, relu=True)

    g2_cm = _mm_scaled(h1, W2, deg, D_OUT // EB)
    agg2 = _sc_aggregate(4, g2_cm.reshape(4 * NP, EB), comb4,
                         zeros128).reshape(4, NP, EB)
    out = _ew_combine(agg2, g2_cm, deg, b2, relu=False)
    return out[:N]
